# Initial kernel scaffold; baseline (speedup 1.0000x reference)
import functools

import jax
import jax.numpy as jnp
from jax import lax
from jax.experimental import pallas as pl
from jax.experimental.pallas import tpu as pltpu
from jax.experimental.pallas import tpu_sc as plsc

N_ENT = 10000
N_REL = 200
INIT_EMB = 128
EMB = 256
HALF = 128
NC = 2
NS = 16
CH = 128
ACC_ROWS = N_ENT + 16
ROWS_PER_TILE = N_ENT // NS
ZERO_PER_TILE = ACC_ROWS // NS


def _edge_counts(n_edges_dir):
    per_tile = -(-n_edges_dir // NS)
    per_tile = -(-per_tile // CH) * CH
    return per_tile, per_tile * NS


def _edge_body(n_chunks, sup_hbm, dst_hbm, src_hbm, typ_hbm, alpha_hbm,
               out_hbm, accum, rows, dstb, srcb, typb, alpexp, gsem, asem):
    c = lax.axis_index("c")
    s = lax.axis_index("s")

    zb = jnp.zeros((16,), jnp.float32)
    for r in range(CH):
        for v in range(HALF // 16):
            rows[r, pl.ds(v * 16, 16)] = zb
    zbase = s * ZERO_PER_TILE
    nfull = ZERO_PER_TILE // CH
    rem = ZERO_PER_TILE - nfull * CH

    def zfill(g):
        pltpu.sync_copy(rows, accum.at[pl.ds(zbase + g * CH, CH)])
        return None
    pl.loop(0, nfull)(zfill)
    if rem:
        pltpu.sync_copy(rows.at[pl.ds(0, rem)],
                        accum.at[pl.ds(zbase + nfull * CH, rem)])
    plsc.subcore_barrier()

    per_tile = n_chunks * CH
    base = s * per_tile

    def chunk_body(g):
        off = base + g * CH
        pltpu.sync_copy(dst_hbm.at[pl.ds(off, CH)], dstb)
        pltpu.sync_copy(src_hbm.at[c, pl.ds(off, CH)], srcb)
        pltpu.sync_copy(typ_hbm.at[pl.ds(off, CH)], typb)
        pltpu.async_copy(sup_hbm.at[srcb], rows, gsem).wait()
        pltpu.async_copy(alpha_hbm.at[typb], alpexp, asem).wait()

        def scale(e):
            asp = alpexp[e]
            for v in range(HALF // 16):
                rows[e, pl.ds(v * 16, 16)] = rows[e, pl.ds(v * 16, 16)] * asp
            return None
        pl.loop(0, CH)(scale)
        pltpu.sync_copy(rows, accum.at[dstb], add=True)
        return None
    pl.loop(0, n_chunks)(chunk_body)

    plsc.subcore_barrier()
    pltpu.sync_copy(accum.at[pl.ds(s * ROWS_PER_TILE, ROWS_PER_TILE)],
                    out_hbm.at[c, pl.ds(s * ROWS_PER_TILE, ROWS_PER_TILE)])


def _make_edge_call(n_chunks):
    mesh = plsc.VectorSubcoreMesh(core_axis_name="c", subcore_axis_name="s",
                                  num_cores=NC, num_subcores=NS)
    return functools.partial(
        pl.kernel,
        out_type=jax.ShapeDtypeStruct((NC, N_ENT, HALF), jnp.float32),
        mesh=mesh,
        scratch_types=[
            pltpu.VMEM_SHARED((ACC_ROWS, HALF), jnp.float32),
            pltpu.VMEM((CH, HALF), jnp.float32),
            pltpu.VMEM((CH,), jnp.int32),
            pltpu.VMEM((CH,), jnp.int32),
            pltpu.VMEM((CH,), jnp.int32),
            pltpu.VMEM((CH, 16), jnp.float32),
            pltpu.SemaphoreType.DMA,
            pltpu.SemaphoreType.DMA,
        ],
    )(functools.partial(_edge_body, n_chunks))


GB = 128


def _gather_body(x3_hbm, rel_tab_hbm, e1_hbm, e2_hbm, rel_hbm,
                 g1_hbm, g2_hbm, gr_hbm, idxb, rowsb, gsem):
    c = lax.axis_index("c")
    s = lax.axis_index("s")
    wid = s * NC + c
    base = wid * GB

    for idx_hbm, tab, out_hbm in ((e1_hbm, x3_hbm, g1_hbm),
                                  (e2_hbm, x3_hbm, g2_hbm),
                                  (rel_hbm, rel_tab_hbm, gr_hbm)):
        pltpu.sync_copy(idx_hbm.at[pl.ds(base, GB)], idxb)
        pltpu.async_copy(tab.at[idxb], rowsb, gsem).wait()
        pltpu.sync_copy(rowsb, out_hbm.at[pl.ds(base, GB)])


def _make_gather_call(batch):
    mesh = plsc.VectorSubcoreMesh(core_axis_name="c", subcore_axis_name="s",
                                  num_cores=NC, num_subcores=NS)
    osd = jax.ShapeDtypeStruct((batch, EMB), jnp.float32)
    return functools.partial(
        pl.kernel,
        out_type=(osd, osd, osd),
        mesh=mesh,
        scratch_types=[
            pltpu.VMEM((GB,), jnp.int32),
            pltpu.VMEM((GB, EMB), jnp.float32),
            pltpu.SemaphoreType.DMA,
        ],
    )(_gather_body)


def _mm_first_body(emb_ref, w_ref, out_ref):
    y = jnp.dot(emb_ref[...], w_ref[...], preferred_element_type=jnp.float32)
    out_ref[0] = y[:, :HALF]
    out_ref[1] = y[:, HALF:]


def _layer_body(agg_ref, b_ref, w_ref, out_ref):
    x = jnp.concatenate([agg_ref[0], agg_ref[1]], axis=1) + b_ref[...]
    m = jnp.mean(x, axis=0)
    v = jnp.mean(jnp.square(x), axis=0) - jnp.square(m)
    y = jnp.tanh((x - m) * lax.rsqrt(v + 1e-5))
    z = jnp.dot(y, w_ref[...], preferred_element_type=jnp.float32)
    out_ref[0] = z[:, :HALF]
    out_ref[1] = z[:, HALF:]


def _final_body(agg_ref, b_ref, out_ref):
    x = jnp.concatenate([agg_ref[0], agg_ref[1]], axis=1) + b_ref[...]
    m = jnp.mean(x, axis=0)
    v = jnp.mean(jnp.square(x), axis=0) - jnp.square(m)
    out_ref[...] = jnp.tanh((x - m) * lax.rsqrt(v + 1e-5))


def _score_body(g1_ref, g2_ref, gr_ref, out_ref):
    srow = jnp.sum(g1_ref[...] * g2_ref[...] * gr_ref[...], axis=1)
    out_ref[...] = 1.0 / (1.0 + jnp.exp(-srow))


def _tc(body, out_shape):
    return pl.pallas_call(body, out_shape=out_shape)


def kernel(e1, rel, e2, X, edge_index, edge_type, num_nodes,
           emb_e, W1, b1, alpha1, W2, b2, alpha2, W3, b3, alpha3, emb_rel):
    n = emb_e.shape[0]
    assert n == N_ENT
    e_dir = 2 * edge_index.shape[1]
    per_tile, e_pad = _edge_counts(e_dir)
    n_chunks = per_tile // CH

    row = edge_index[0].astype(jnp.int32)
    col = edge_index[1].astype(jnp.int32)
    et = edge_type.astype(jnp.int32)
    pad = e_pad - e_dir
    dst_dir = jnp.pad(jnp.concatenate([row, col]), (0, pad),
                      constant_values=N_ENT + 8)
    src_dir = jnp.pad(jnp.concatenate([col, row]), (0, pad))
    typ_dir = jnp.pad(jnp.concatenate([et, et]), (0, pad))
    src2 = jnp.stack([src_dir, src_dir + N_ENT])
    alpha16 = [jnp.tile(a.astype(jnp.float32), (1, 16)) for a in
               (alpha1, alpha2, alpha3)]

    edge_call = _make_edge_call(n_chunks)

    def sc_pass(sup, a16):
        return edge_call(sup.reshape(2 * N_ENT, HALF), dst_dir, src2,
                         typ_dir, a16)

    sup = _tc(_mm_first_body,
              jax.ShapeDtypeStruct((2, N_ENT, HALF), jnp.float32))(emb_e, W1)
    agg = sc_pass(sup, alpha16[0])
    sup = _tc(_layer_body,
              jax.ShapeDtypeStruct((2, N_ENT, HALF), jnp.float32))(agg, b1, W2)
    agg = sc_pass(sup, alpha16[1])
    sup = _tc(_layer_body,
              jax.ShapeDtypeStruct((2, N_ENT, HALF), jnp.float32))(agg, b2, W3)
    agg = sc_pass(sup, alpha16[2])
    x3 = _tc(_final_body,
             jax.ShapeDtypeStruct((N_ENT, EMB), jnp.float32))(agg, b3)

    batch = e1.shape[0]
    g1, g2, gr = _make_gather_call(batch)(
        x3, emb_rel, e1.astype(jnp.int32), e2.astype(jnp.int32),
        rel.astype(jnp.int32))
    pred = _tc(_score_body,
               jax.ShapeDtypeStruct((batch,), jnp.float32))(g1, g2, gr)
    return pred


# speedup vs baseline: 3.8552x; 3.8552x over previous
import functools

import jax
import jax.numpy as jnp
from jax import lax
from jax.experimental import pallas as pl
from jax.experimental.pallas import tpu as pltpu
from jax.experimental.pallas import tpu_sc as plsc

N_ENT = 10000
N_REL = 200
INIT_EMB = 128
EMB = 256
HALF = 128
NC = 2
NS = 16
CH = 128
ACC_ROWS = N_ENT + 16
STRIDE = 624
ZSPAN = ACC_ROWS - 15 * STRIDE
TAIL = N_ENT - NS * STRIDE


def _edge_counts(n_edges_dir):
    per_tile = -(-n_edges_dir // NS)
    per_tile = -(-per_tile // CH) * CH
    return per_tile, per_tile * NS


def _edge_body(n_chunks, sup_hbm, dst_hbm, src_hbm, typ_hbm, alpha_hbm,
               out_hbm, accum, rows, dstb, srcb, typb, alpha_v, alp16b, gsem):
    c = lax.axis_index("c")
    s = lax.axis_index("s")
    pltpu.sync_copy(alpha_hbm, alpha_v)

    zb = jnp.zeros((16,), jnp.float32)
    for r in range(CH):
        for v in range(HALF // 16):
            rows[r, pl.ds(v * 16, 16)] = zb
    zbase = s * STRIDE
    nfull = ZSPAN // CH
    rem = ZSPAN - nfull * CH

    def zfill(g):
        pltpu.sync_copy(rows, accum.at[pl.ds(zbase + g * CH, CH)])
        return None
    pl.loop(0, nfull)(zfill)
    if rem:
        pltpu.sync_copy(rows.at[pl.ds(0, rem)],
                        accum.at[pl.ds(zbase + nfull * CH, rem)])
    plsc.subcore_barrier()

    per_tile = n_chunks * CH
    base = s * per_tile

    def chunk_body(g):
        off = base + g * CH
        pltpu.sync_copy(dst_hbm.at[pl.ds(off, CH)], dstb)
        pltpu.sync_copy(src_hbm.at[c, pl.ds(off, CH)], srcb)
        pltpu.sync_copy(typ_hbm.at[pl.ds(off, CH)], typb)
        pltpu.async_copy(sup_hbm.at[srcb], rows, gsem).wait()

        def grp(g2):
            tv = typb[pl.ds(g2 * 16, 16)]
            alp16b[...] = plsc.load_gather(alpha_v, [tv])

            def inner(j):
                asp = plsc.load_gather(
                    alp16b, [jnp.full((16,), j, jnp.int32)])
                e = g2 * 16 + j
                for v in range(HALF // 16):
                    rows[e, pl.ds(v * 16, 16)] = (
                        rows[e, pl.ds(v * 16, 16)] * asp)
                return None
            pl.loop(0, 16)(inner)
            return None
        pl.loop(0, CH // 16)(grp)
        pltpu.sync_copy(rows, accum.at[dstb], add=True)
        return None
    pl.loop(0, n_chunks)(chunk_body)

    plsc.subcore_barrier()
    pltpu.sync_copy(accum.at[pl.ds(s * STRIDE, STRIDE)],
                    out_hbm.at[c, pl.ds(s * STRIDE, STRIDE)])

    @pl.when(s == 0)
    def _copy_tail():
        pltpu.sync_copy(accum.at[pl.ds(NS * STRIDE, TAIL)],
                        out_hbm.at[c, pl.ds(NS * STRIDE, TAIL)])


def _make_edge_call(n_chunks):
    mesh = plsc.VectorSubcoreMesh(core_axis_name="c", subcore_axis_name="s",
                                  num_cores=NC, num_subcores=NS)
    return functools.partial(
        pl.kernel,
        out_type=jax.ShapeDtypeStruct((NC, N_ENT, HALF), jnp.float32),
        mesh=mesh,
        compiler_params=pltpu.CompilerParams(needs_layout_passes=False),
        scratch_types=[
            pltpu.VMEM_SHARED((ACC_ROWS, HALF), jnp.float32),
            pltpu.VMEM((CH, HALF), jnp.float32),
            pltpu.VMEM((CH,), jnp.int32),
            pltpu.VMEM((CH,), jnp.int32),
            pltpu.VMEM((CH,), jnp.int32),
            pltpu.VMEM((208,), jnp.float32),
            pltpu.VMEM((16,), jnp.float32),
            pltpu.SemaphoreType.DMA,
        ],
    )(functools.partial(_edge_body, n_chunks))


GB = 128


def _gather_body(x3_hbm, rel_tab_hbm, e1_hbm, e2_hbm, rel_hbm,
                 g1_hbm, g2_hbm, gr_hbm, idxb, rowsb, gsem):
    c = lax.axis_index("c")
    s = lax.axis_index("s")
    wid = s * NC + c
    base = wid * GB

    for idx_hbm, tab, out_hbm in ((e1_hbm, x3_hbm, g1_hbm),
                                  (e2_hbm, x3_hbm, g2_hbm),
                                  (rel_hbm, rel_tab_hbm, gr_hbm)):
        pltpu.sync_copy(idx_hbm.at[pl.ds(base, GB)], idxb)
        pltpu.async_copy(tab.at[idxb], rowsb, gsem).wait()
        pltpu.sync_copy(rowsb, out_hbm.at[pl.ds(base, GB)])


def _make_gather_call(batch):
    mesh = plsc.VectorSubcoreMesh(core_axis_name="c", subcore_axis_name="s",
                                  num_cores=NC, num_subcores=NS)
    osd = jax.ShapeDtypeStruct((batch, EMB), jnp.float32)
    return functools.partial(
        pl.kernel,
        out_type=(osd, osd, osd),
        mesh=mesh,
        scratch_types=[
            pltpu.VMEM((GB,), jnp.int32),
            pltpu.VMEM((GB, EMB), jnp.float32),
            pltpu.SemaphoreType.DMA,
        ],
    )(_gather_body)


def _mm_first_body(emb_ref, w_ref, out_ref):
    y = jnp.dot(emb_ref[...], w_ref[...], preferred_element_type=jnp.float32)
    out_ref[0] = y[:, :HALF]
    out_ref[1] = y[:, HALF:]


def _layer_body(agg_ref, b_ref, w_ref, out_ref):
    x = jnp.concatenate([agg_ref[0], agg_ref[1]], axis=1) + b_ref[...]
    m = jnp.mean(x, axis=0)
    v = jnp.mean(jnp.square(x), axis=0) - jnp.square(m)
    y = jnp.tanh((x - m) * lax.rsqrt(v + 1e-5))
    z = jnp.dot(y, w_ref[...], preferred_element_type=jnp.float32)
    out_ref[0] = z[:, :HALF]
    out_ref[1] = z[:, HALF:]


def _final_body(agg_ref, b_ref, out_ref):
    x = jnp.concatenate([agg_ref[0], agg_ref[1]], axis=1) + b_ref[...]
    m = jnp.mean(x, axis=0)
    v = jnp.mean(jnp.square(x), axis=0) - jnp.square(m)
    out_ref[...] = jnp.tanh((x - m) * lax.rsqrt(v + 1e-5))


def _score_body(g1_ref, g2_ref, gr_ref, out_ref):
    srow = jnp.sum(g1_ref[...] * g2_ref[...] * gr_ref[...], axis=1)
    out_ref[...] = 1.0 / (1.0 + jnp.exp(-srow))


def _tc(body, out_shape):
    return pl.pallas_call(body, out_shape=out_shape)


def kernel(e1, rel, e2, X, edge_index, edge_type, num_nodes,
           emb_e, W1, b1, alpha1, W2, b2, alpha2, W3, b3, alpha3, emb_rel):
    n = emb_e.shape[0]
    assert n == N_ENT
    e_dir = 2 * edge_index.shape[1]
    per_tile, e_pad = _edge_counts(e_dir)
    n_chunks = per_tile // CH

    row = edge_index[0].astype(jnp.int32)
    col = edge_index[1].astype(jnp.int32)
    et = edge_type.astype(jnp.int32)
    pad = e_pad - e_dir
    dst_dir = jnp.pad(jnp.concatenate([row, col]), (0, pad),
                      constant_values=N_ENT + 8)
    src_dir = jnp.pad(jnp.concatenate([col, row]), (0, pad))
    typ_dir = jnp.pad(jnp.concatenate([et, et]), (0, pad))
    src2 = jnp.stack([src_dir, src_dir + N_ENT])
    alpha_pad = [jnp.pad(a.astype(jnp.float32)[:, 0], (0, 208 - a.shape[0]))
                 for a in (alpha1, alpha2, alpha3)]

    edge_call = _make_edge_call(n_chunks)

    def sc_pass(sup, a16):
        return edge_call(sup.reshape(2 * N_ENT, HALF), dst_dir, src2,
                         typ_dir, a16)

    sup = _tc(_mm_first_body,
              jax.ShapeDtypeStruct((2, N_ENT, HALF), jnp.float32))(emb_e, W1)
    agg = sc_pass(sup, alpha_pad[0])
    sup = _tc(_layer_body,
              jax.ShapeDtypeStruct((2, N_ENT, HALF), jnp.float32))(agg, b1, W2)
    agg = sc_pass(sup, alpha_pad[1])
    sup = _tc(_layer_body,
              jax.ShapeDtypeStruct((2, N_ENT, HALF), jnp.float32))(agg, b2, W3)
    agg = sc_pass(sup, alpha_pad[2])
    x3 = _tc(_final_body,
             jax.ShapeDtypeStruct((N_ENT, EMB), jnp.float32))(agg, b3)

    batch = e1.shape[0]
    g1, g2, gr = _make_gather_call(batch)(
        x3, emb_rel, e1.astype(jnp.int32), e2.astype(jnp.int32),
        rel.astype(jnp.int32))
    pred = _tc(_score_body,
               jax.ShapeDtypeStruct((batch,), jnp.float32))(g1, g2, gr)
    return pred


# speedup vs baseline: 5.7562x; 1.4931x over previous
import functools

import jax
import jax.numpy as jnp
from jax import lax
from jax.experimental import pallas as pl
from jax.experimental.pallas import tpu as pltpu
from jax.experimental.pallas import tpu_sc as plsc

N_ENT = 10000
N_REL = 200
INIT_EMB = 128
EMB = 256
HALF = 128
NC = 2
NS = 16
CH = 64
ACC_ROWS = N_ENT + 16
STRIDE = 624
ZSPAN = ACC_ROWS - 15 * STRIDE
TAIL = N_ENT - NS * STRIDE


def _edge_counts(n_edges_dir):
    per_tile = -(-n_edges_dir // NS)
    per_tile = -(-per_tile // (2 * CH)) * (2 * CH)
    return per_tile, per_tile * NS


def _splat(vec, j):
    return lax.gather(
        vec, jnp.full((16, 1), j, jnp.int32),
        lax.GatherDimensionNumbers(offset_dims=(), collapsed_slice_dims=(0,),
                                   start_index_map=(0,)),
        (1,), mode=lax.GatherScatterMode.PROMISE_IN_BOUNDS)


def _edge_body(n_chunks, sup_hbm, st_hbm, dst_hbm, alpha_hbm,
               out_hbm, accum, rows, srows, stb, dstb, alpha_v,
               gsem0, gsem1, ssem0, ssem1):
    c = lax.axis_index("c")
    s = lax.axis_index("s")
    gsems = (gsem0, gsem1)
    ssems = (ssem0, ssem1)
    pltpu.sync_copy(alpha_hbm, alpha_v)

    zb = jnp.zeros((16,), jnp.float32)
    for r in range(CH):
        for v in range(HALF // 16):
            rows[0, r, pl.ds(v * 16, 16)] = zb
    zbase = s * STRIDE
    nfull = ZSPAN // CH
    rem = ZSPAN - nfull * CH

    def zfill(g):
        pltpu.sync_copy(rows.at[0], accum.at[pl.ds(zbase + g * CH, CH)])
        return None
    pl.loop(0, nfull)(zfill)
    if rem:
        pltpu.sync_copy(rows.at[0, pl.ds(0, rem)],
                        accum.at[pl.ds(zbase + nfull * CH, rem)])
    plsc.subcore_barrier()

    per_tile = n_chunks * CH
    base_st = s * (2 * per_tile)
    base_d = s * per_tile

    def start_gather(g, b):
        pltpu.sync_copy(st_hbm.at[c, pl.ds(base_st + g * (2 * CH), 2 * CH)],
                        stb.at[b])
        pltpu.async_copy(
            sup_hbm.at[stb.at[b, pl.ds(0, CH)]], rows.at[b], gsems[b])

    start_gather(0, 0)
    start_gather(1, 1)

    def pair_body(h):
        for b in range(2):
            g = h * 2 + b
            pltpu.make_async_copy(sup_hbm.at[stb.at[b, pl.ds(0, CH)]],
                                  rows.at[b], gsems[b]).wait()
            @pl.when(h > 0)
            def _drain():
                pltpu.make_async_copy(srows.at[b], accum.at[dstb.at[b]],
                                      ssems[b]).wait()
            pltpu.sync_copy(dst_hbm.at[c, pl.ds(base_d + g * CH, CH)],
                            dstb.at[b])

            def grp(g2):
                tv = stb[b, pl.ds(CH + g2 * 16, 16)]
                a16 = plsc.load_gather(alpha_v, [tv])
                for j in range(16):
                    asp = _splat(a16, j)
                    e = g2 * 16 + j
                    for v in range(HALF // 16):
                        srows[b, e, pl.ds(v * 16, 16)] = (
                            rows[b, e, pl.ds(v * 16, 16)] * asp)
                return None
            pl.loop(0, CH // 16)(grp)

            pltpu.async_copy(srows.at[b], accum.at[dstb.at[b]], ssems[b],
                             add=True)
            @pl.when(g + 2 < n_chunks)
            def _prefetch():
                start_gather(g + 2, b)
        return None
    pl.loop(0, n_chunks // 2)(pair_body)

    for b in range(2):
        pltpu.make_async_copy(srows.at[b], accum.at[dstb.at[b]],
                              ssems[b]).wait()

    plsc.subcore_barrier()
    pltpu.sync_copy(accum.at[pl.ds(s * STRIDE, STRIDE)],
                    out_hbm.at[c, pl.ds(s * STRIDE, STRIDE)])

    @pl.when(s == 0)
    def _copy_tail():
        pltpu.sync_copy(accum.at[pl.ds(NS * STRIDE, TAIL)],
                        out_hbm.at[c, pl.ds(NS * STRIDE, TAIL)])


def _make_edge_call(n_chunks):
    mesh = plsc.VectorSubcoreMesh(core_axis_name="c", subcore_axis_name="s",
                                  num_cores=NC, num_subcores=NS)
    return functools.partial(
        pl.kernel,
        out_type=jax.ShapeDtypeStruct((NC, N_ENT, HALF), jnp.float32),
        mesh=mesh,
        compiler_params=pltpu.CompilerParams(needs_layout_passes=False),
        scratch_types=[
            pltpu.VMEM_SHARED((ACC_ROWS, HALF), jnp.float32),
            pltpu.VMEM((2, CH, HALF), jnp.float32),
            pltpu.VMEM((2, CH, HALF), jnp.float32),
            pltpu.VMEM((2, 2 * CH), jnp.int32),
            pltpu.VMEM((2, CH), jnp.int32),
            pltpu.VMEM((208,), jnp.float32),
            pltpu.SemaphoreType.DMA,
            pltpu.SemaphoreType.DMA,
            pltpu.SemaphoreType.DMA,
            pltpu.SemaphoreType.DMA,
        ],
    )(functools.partial(_edge_body, n_chunks))


GB = 128


def _gather_body(x3_hbm, rel_tab_hbm, e1_hbm, e2_hbm, rel_hbm,
                 g1_hbm, g2_hbm, gr_hbm, idxb, rowsb, gsem):
    c = lax.axis_index("c")
    s = lax.axis_index("s")
    wid = s * NC + c
    base = wid * GB

    for idx_hbm, tab, out_hbm in ((e1_hbm, x3_hbm, g1_hbm),
                                  (e2_hbm, x3_hbm, g2_hbm),
                                  (rel_hbm, rel_tab_hbm, gr_hbm)):
        pltpu.sync_copy(idx_hbm.at[pl.ds(base, GB)], idxb)
        pltpu.async_copy(tab.at[idxb], rowsb, gsem).wait()
        pltpu.sync_copy(rowsb, out_hbm.at[pl.ds(base, GB)])


def _make_gather_call(batch):
    mesh = plsc.VectorSubcoreMesh(core_axis_name="c", subcore_axis_name="s",
                                  num_cores=NC, num_subcores=NS)
    osd = jax.ShapeDtypeStruct((batch, EMB), jnp.float32)
    return functools.partial(
        pl.kernel,
        out_type=(osd, osd, osd),
        mesh=mesh,
        scratch_types=[
            pltpu.VMEM((GB,), jnp.int32),
            pltpu.VMEM((GB, EMB), jnp.float32),
            pltpu.SemaphoreType.DMA,
        ],
    )(_gather_body)


def _mm_first_body(emb_ref, w_ref, out_ref):
    y = jnp.dot(emb_ref[...], w_ref[...], preferred_element_type=jnp.float32)
    out_ref[0] = y[:, :HALF]
    out_ref[1] = y[:, HALF:]


def _layer_body(agg_ref, b_ref, w_ref, out_ref):
    x = jnp.concatenate([agg_ref[0], agg_ref[1]], axis=1) + b_ref[...]
    m = jnp.mean(x, axis=0)
    v = jnp.mean(jnp.square(x), axis=0) - jnp.square(m)
    y = jnp.tanh((x - m) * lax.rsqrt(v + 1e-5))
    z = jnp.dot(y, w_ref[...], preferred_element_type=jnp.float32)
    out_ref[0] = z[:, :HALF]
    out_ref[1] = z[:, HALF:]


def _final_body(agg_ref, b_ref, out_ref):
    x = jnp.concatenate([agg_ref[0], agg_ref[1]], axis=1) + b_ref[...]
    m = jnp.mean(x, axis=0)
    v = jnp.mean(jnp.square(x), axis=0) - jnp.square(m)
    out_ref[...] = jnp.tanh((x - m) * lax.rsqrt(v + 1e-5))


def _score_body(g1_ref, g2_ref, gr_ref, out_ref):
    srow = jnp.sum(g1_ref[...] * g2_ref[...] * gr_ref[...], axis=1)
    out_ref[...] = 1.0 / (1.0 + jnp.exp(-srow))


def _tc(body, out_shape):
    return pl.pallas_call(body, out_shape=out_shape)


def kernel(e1, rel, e2, X, edge_index, edge_type, num_nodes,
           emb_e, W1, b1, alpha1, W2, b2, alpha2, W3, b3, alpha3, emb_rel):
    n = emb_e.shape[0]
    assert n == N_ENT
    e_dir = 2 * edge_index.shape[1]
    per_tile, e_pad = _edge_counts(e_dir)
    n_chunks = per_tile // CH

    row = edge_index[0].astype(jnp.int32)
    col = edge_index[1].astype(jnp.int32)
    et = edge_type.astype(jnp.int32)
    pad = e_pad - e_dir
    dst_dir = jnp.pad(jnp.concatenate([row, col]), (0, pad),
                      constant_values=N_ENT + 8)
    src_dir = jnp.pad(jnp.concatenate([col, row]), (0, pad))
    typ_dir = jnp.pad(jnp.concatenate([et, et]), (0, pad))
    src2 = jnp.stack([src_dir, src_dir + N_ENT])
    src_blk = src2.reshape(NC, NS, n_chunks, CH)
    typ_blk = jnp.broadcast_to(typ_dir.reshape(1, NS, n_chunks, CH),
                               (NC, NS, n_chunks, CH))
    st_pack = jnp.concatenate([src_blk, typ_blk], axis=3).reshape(NC, -1)
    dst2 = jnp.broadcast_to(dst_dir.reshape(1, NS * n_chunks * CH),
                            (NC, NS * n_chunks * CH))
    alpha_pad = [jnp.pad(a.astype(jnp.float32)[:, 0], (0, 208 - a.shape[0]))
                 for a in (alpha1, alpha2, alpha3)]

    edge_call = _make_edge_call(n_chunks)

    def sc_pass(sup, a16):
        return edge_call(sup.reshape(2 * N_ENT, HALF), st_pack, dst2, a16)

    sup = _tc(_mm_first_body,
              jax.ShapeDtypeStruct((2, N_ENT, HALF), jnp.float32))(emb_e, W1)
    agg = sc_pass(sup, alpha_pad[0])
    sup = _tc(_layer_body,
              jax.ShapeDtypeStruct((2, N_ENT, HALF), jnp.float32))(agg, b1, W2)
    agg = sc_pass(sup, alpha_pad[1])
    sup = _tc(_layer_body,
              jax.ShapeDtypeStruct((2, N_ENT, HALF), jnp.float32))(agg, b2, W3)
    agg = sc_pass(sup, alpha_pad[2])
    x3 = _tc(_final_body,
             jax.ShapeDtypeStruct((N_ENT, EMB), jnp.float32))(agg, b3)

    batch = e1.shape[0]
    g1, g2, gr = _make_gather_call(batch)(
        x3, emb_rel, e1.astype(jnp.int32), e2.astype(jnp.int32),
        rel.astype(jnp.int32))
    pred = _tc(_score_body,
               jax.ShapeDtypeStruct((batch,), jnp.float32))(g1, g2, gr)
    return pred


# speedup vs baseline: 6.0659x; 1.0538x over previous
import functools

import jax
import jax.numpy as jnp
from jax import lax
from jax.experimental import pallas as pl
from jax.experimental.pallas import tpu as pltpu
from jax.experimental.pallas import tpu_sc as plsc

N_ENT = 10000
N_REL = 200
INIT_EMB = 128
EMB = 256
HALF = 128
NC = 2
NS = 16
CH = 64
ACC_ROWS = N_ENT + 16
STRIDE = 624
ZSPAN = ACC_ROWS - 15 * STRIDE
TAIL = N_ENT - NS * STRIDE


def _edge_counts(n_edges_dir):
    per_tile = -(-n_edges_dir // NS)
    per_tile = -(-per_tile // (6 * CH)) * (6 * CH)
    return per_tile, per_tile * NS


def _splat(vec, j):
    return lax.gather(
        vec, jnp.full((16, 1), j, jnp.int32),
        lax.GatherDimensionNumbers(offset_dims=(), collapsed_slice_dims=(0,),
                                   start_index_map=(0,)),
        (1,), mode=lax.GatherScatterMode.PROMISE_IN_BOUNDS)


def _edge_body(n_chunks, sup_hbm, st_hbm, dst_hbm, alpha_hbm,
               out_hbm, accum, rows, srows, stb, dstb, alpha_v,
               gsem0, gsem1, ssem0, ssem1, tsem0, tsem1, tsem2, dsem0, dsem1):
    c = lax.axis_index("c")
    s = lax.axis_index("s")
    gsems = (gsem0, gsem1)
    ssems = (ssem0, ssem1)
    tsems = (tsem0, tsem1, tsem2)
    dsems = (dsem0, dsem1)
    pltpu.sync_copy(alpha_hbm, alpha_v)

    zb = jnp.zeros((16,), jnp.float32)
    for r in range(CH):
        for v in range(HALF // 16):
            rows[0, r, pl.ds(v * 16, 16)] = zb
    zbase = s * STRIDE
    nfull = ZSPAN // CH
    rem = ZSPAN - nfull * CH

    def zfill(g):
        pltpu.sync_copy(rows.at[0], accum.at[pl.ds(zbase + g * CH, CH)])
        return None
    pl.loop(0, nfull)(zfill)
    if rem:
        pltpu.sync_copy(rows.at[0, pl.ds(0, rem)],
                        accum.at[pl.ds(zbase + nfull * CH, rem)])
    plsc.subcore_barrier()

    per_tile = n_chunks * CH
    base_st = s * (2 * per_tile)
    base_d = s * per_tile

    def start_st(g, r):
        pltpu.async_copy(st_hbm.at[c, pl.ds(base_st + g * (2 * CH), 2 * CH)],
                         stb.at[r], tsems[r])

    def wait_st(g, r):
        pltpu.make_async_copy(st_hbm.at[c, pl.ds(base_st + g * (2 * CH),
                                                 2 * CH)],
                              stb.at[r], tsems[r]).wait()

    def start_gather(g, r, b):
        pltpu.async_copy(
            sup_hbm.at[stb.at[r, pl.ds(0, CH)]], rows.at[b], gsems[b])

    start_st(0, 0)
    start_st(1, 1)
    start_st(2, 2)
    wait_st(0, 0)
    start_gather(0, 0, 0)
    wait_st(1, 1)
    start_gather(1, 1, 1)

    def six_body(h):
        for k in range(6):
            g = h * 6 + k
            b = k % 2
            r = k % 3
            pltpu.make_async_copy(sup_hbm.at[stb.at[r, pl.ds(0, CH)]],
                                  rows.at[b], gsems[b]).wait()
            @pl.when(g >= 2)
            def _drain():
                pltpu.make_async_copy(srows.at[b], accum.at[dstb.at[b]],
                                      ssems[b]).wait()
            pltpu.async_copy(dst_hbm.at[c, pl.ds(base_d + g * CH, CH)],
                            dstb.at[b], dsems[b])

            def grp(g2):
                tv = stb[r, pl.ds(CH + g2 * 16, 16)]
                a16 = plsc.load_gather(alpha_v, [tv])
                for j in range(16):
                    asp = _splat(a16, j)
                    e = g2 * 16 + j
                    for v in range(HALF // 16):
                        srows[b, e, pl.ds(v * 16, 16)] = (
                            rows[b, e, pl.ds(v * 16, 16)] * asp)
                return None
            pl.loop(0, CH // 16)(grp)

            @pl.when(g + 3 < n_chunks)
            def _st_prefetch():
                start_st(g + 3, r)
            @pl.when(g + 2 < n_chunks)
            def _g_prefetch():
                wait_st(g + 2, (k + 2) % 3)
                start_gather(g + 2, (k + 2) % 3, b)
            pltpu.make_async_copy(dst_hbm.at[c, pl.ds(base_d + g * CH, CH)],
                                  dstb.at[b], dsems[b]).wait()
            pltpu.async_copy(srows.at[b], accum.at[dstb.at[b]], ssems[b],
                             add=True)
        return None
    pl.loop(0, n_chunks // 6)(six_body)

    for b in range(2):
        pltpu.make_async_copy(srows.at[b], accum.at[dstb.at[b]],
                              ssems[b]).wait()

    plsc.subcore_barrier()
    pltpu.sync_copy(accum.at[pl.ds(s * STRIDE, STRIDE)],
                    out_hbm.at[c, pl.ds(s * STRIDE, STRIDE)])

    @pl.when(s == 0)
    def _copy_tail():
        pltpu.sync_copy(accum.at[pl.ds(NS * STRIDE, TAIL)],
                        out_hbm.at[c, pl.ds(NS * STRIDE, TAIL)])


def _make_edge_call(n_chunks):
    mesh = plsc.VectorSubcoreMesh(core_axis_name="c", subcore_axis_name="s",
                                  num_cores=NC, num_subcores=NS)
    return functools.partial(
        pl.kernel,
        out_type=jax.ShapeDtypeStruct((NC, N_ENT, HALF), jnp.float32),
        mesh=mesh,
        compiler_params=pltpu.CompilerParams(needs_layout_passes=False),
        scratch_types=[
            pltpu.VMEM_SHARED((ACC_ROWS, HALF), jnp.float32),
            pltpu.VMEM((2, CH, HALF), jnp.float32),
            pltpu.VMEM((2, CH, HALF), jnp.float32),
            pltpu.VMEM((3, 2 * CH), jnp.int32),
            pltpu.VMEM((2, CH), jnp.int32),
            pltpu.VMEM((208,), jnp.float32),
            pltpu.SemaphoreType.DMA,
            pltpu.SemaphoreType.DMA,
            pltpu.SemaphoreType.DMA,
            pltpu.SemaphoreType.DMA,
            pltpu.SemaphoreType.DMA,
            pltpu.SemaphoreType.DMA,
            pltpu.SemaphoreType.DMA,
            pltpu.SemaphoreType.DMA,
            pltpu.SemaphoreType.DMA,
        ],
    )(functools.partial(_edge_body, n_chunks))


GB = 128


def _gather_body(x3_hbm, rel_tab_hbm, e1_hbm, e2_hbm, rel_hbm,
                 g1_hbm, g2_hbm, gr_hbm, idxb, rowsb, gsem):
    c = lax.axis_index("c")
    s = lax.axis_index("s")
    wid = s * NC + c
    base = wid * GB

    for idx_hbm, tab, out_hbm in ((e1_hbm, x3_hbm, g1_hbm),
                                  (e2_hbm, x3_hbm, g2_hbm),
                                  (rel_hbm, rel_tab_hbm, gr_hbm)):
        pltpu.sync_copy(idx_hbm.at[pl.ds(base, GB)], idxb)
        pltpu.async_copy(tab.at[idxb], rowsb, gsem).wait()
        pltpu.sync_copy(rowsb, out_hbm.at[pl.ds(base, GB)])


def _make_gather_call(batch):
    mesh = plsc.VectorSubcoreMesh(core_axis_name="c", subcore_axis_name="s",
                                  num_cores=NC, num_subcores=NS)
    osd = jax.ShapeDtypeStruct((batch, EMB), jnp.float32)
    return functools.partial(
        pl.kernel,
        out_type=(osd, osd, osd),
        mesh=mesh,
        scratch_types=[
            pltpu.VMEM((GB,), jnp.int32),
            pltpu.VMEM((GB, EMB), jnp.float32),
            pltpu.SemaphoreType.DMA,
        ],
    )(_gather_body)


def _mm_first_body(emb_ref, w_ref, out_ref):
    y = jnp.dot(emb_ref[...], w_ref[...], preferred_element_type=jnp.float32)
    out_ref[0] = y[:, :HALF]
    out_ref[1] = y[:, HALF:]


def _layer_body(agg_ref, b_ref, w_ref, out_ref):
    x = jnp.concatenate([agg_ref[0], agg_ref[1]], axis=1) + b_ref[...]
    m = jnp.mean(x, axis=0)
    v = jnp.mean(jnp.square(x), axis=0) - jnp.square(m)
    y = jnp.tanh((x - m) * lax.rsqrt(v + 1e-5))
    z = jnp.dot(y, w_ref[...], preferred_element_type=jnp.float32)
    out_ref[0] = z[:, :HALF]
    out_ref[1] = z[:, HALF:]


def _final_body(agg_ref, b_ref, out_ref):
    x = jnp.concatenate([agg_ref[0], agg_ref[1]], axis=1) + b_ref[...]
    m = jnp.mean(x, axis=0)
    v = jnp.mean(jnp.square(x), axis=0) - jnp.square(m)
    out_ref[...] = jnp.tanh((x - m) * lax.rsqrt(v + 1e-5))


def _score_body(g1_ref, g2_ref, gr_ref, out_ref):
    srow = jnp.sum(g1_ref[...] * g2_ref[...] * gr_ref[...], axis=1)
    out_ref[...] = 1.0 / (1.0 + jnp.exp(-srow))


def _tc(body, out_shape):
    return pl.pallas_call(body, out_shape=out_shape)


def kernel(e1, rel, e2, X, edge_index, edge_type, num_nodes,
           emb_e, W1, b1, alpha1, W2, b2, alpha2, W3, b3, alpha3, emb_rel):
    n = emb_e.shape[0]
    assert n == N_ENT
    e_dir = 2 * edge_index.shape[1]
    per_tile, e_pad = _edge_counts(e_dir)
    n_chunks = per_tile // CH

    row = edge_index[0].astype(jnp.int32)
    col = edge_index[1].astype(jnp.int32)
    et = edge_type.astype(jnp.int32)
    pad = e_pad - e_dir
    dst_dir = jnp.pad(jnp.concatenate([row, col]), (0, pad),
                      constant_values=N_ENT + 8)
    src_dir = jnp.pad(jnp.concatenate([col, row]), (0, pad))
    typ_dir = jnp.pad(jnp.concatenate([et, et]), (0, pad))
    src2 = jnp.stack([src_dir, src_dir + N_ENT])
    src_blk = src2.reshape(NC, NS, n_chunks, CH)
    typ_blk = jnp.broadcast_to(typ_dir.reshape(1, NS, n_chunks, CH),
                               (NC, NS, n_chunks, CH))
    st_pack = jnp.concatenate([src_blk, typ_blk], axis=3).reshape(NC, -1)
    dst2 = jnp.broadcast_to(dst_dir.reshape(1, NS * n_chunks * CH),
                            (NC, NS * n_chunks * CH))
    alpha_pad = [jnp.pad(a.astype(jnp.float32)[:, 0], (0, 208 - a.shape[0]))
                 for a in (alpha1, alpha2, alpha3)]

    edge_call = _make_edge_call(n_chunks)

    def sc_pass(sup, a16):
        return edge_call(sup.reshape(2 * N_ENT, HALF), st_pack, dst2, a16)

    sup = _tc(_mm_first_body,
              jax.ShapeDtypeStruct((2, N_ENT, HALF), jnp.float32))(emb_e, W1)
    agg = sc_pass(sup, alpha_pad[0])
    sup = _tc(_layer_body,
              jax.ShapeDtypeStruct((2, N_ENT, HALF), jnp.float32))(agg, b1, W2)
    agg = sc_pass(sup, alpha_pad[1])
    sup = _tc(_layer_body,
              jax.ShapeDtypeStruct((2, N_ENT, HALF), jnp.float32))(agg, b2, W3)
    agg = sc_pass(sup, alpha_pad[2])
    x3 = _tc(_final_body,
             jax.ShapeDtypeStruct((N_ENT, EMB), jnp.float32))(agg, b3)

    batch = e1.shape[0]
    g1, g2, gr = _make_gather_call(batch)(
        x3, emb_rel, e1.astype(jnp.int32), e2.astype(jnp.int32),
        rel.astype(jnp.int32))
    pred = _tc(_score_body,
               jax.ShapeDtypeStruct((batch,), jnp.float32))(g1, g2, gr)
    return pred


# speedup vs baseline: 6.2901x; 1.0370x over previous
import functools

import jax
import jax.numpy as jnp
from jax import lax
from jax.experimental import pallas as pl
from jax.experimental.pallas import tpu as pltpu
from jax.experimental.pallas import tpu_sc as plsc

N_ENT = 10000
N_REL = 200
INIT_EMB = 128
EMB = 256
HALF = 128
NC = 2
NS = 16
CH = 96
ACC_ROWS = N_ENT
STRIDE = 624
ZSPAN = ACC_ROWS - 15 * STRIDE
TAIL = N_ENT - NS * STRIDE


def _edge_counts(n_edges_dir):
    per_tile = -(-n_edges_dir // NS)
    per_tile = -(-per_tile // (6 * CH)) * (6 * CH)
    return per_tile, per_tile * NS


def _splat(vec, j):
    return lax.gather(
        vec, jnp.full((16, 1), j, jnp.int32),
        lax.GatherDimensionNumbers(offset_dims=(), collapsed_slice_dims=(0,),
                                   start_index_map=(0,)),
        (1,), mode=lax.GatherScatterMode.PROMISE_IN_BOUNDS)


def _edge_body(n_chunks, sup_hbm, st_hbm, dst_hbm, alpha_hbm,
               out_hbm, accum, rows, srows, stb0, stb1, stb2, dstb0, dstb1,
               alpha_v, gsem0, gsem1, ssem, tsem0, tsem1, tsem2, dsem0, dsem1):
    c = lax.axis_index("c")
    s = lax.axis_index("s")
    gsems = (gsem0, gsem1)
    tsems = (tsem0, tsem1, tsem2)
    dsems = (dsem0, dsem1)
    stbs = (stb0, stb1, stb2)
    dstbs = (dstb0, dstb1)
    pltpu.sync_copy(alpha_hbm, alpha_v)

    zb = jnp.zeros((16,), jnp.float32)
    for r in range(CH):
        for v in range(HALF // 16):
            rows[0, r, pl.ds(v * 16, 16)] = zb
    zbase = s * STRIDE
    nfull = ZSPAN // CH
    rem = ZSPAN - nfull * CH

    def zfill(g):
        pltpu.sync_copy(rows.at[0], accum.at[pl.ds(zbase + g * CH, CH)])
        return None
    pl.loop(0, nfull)(zfill)
    if rem:
        pltpu.sync_copy(rows.at[0, pl.ds(0, rem)],
                        accum.at[pl.ds(zbase + nfull * CH, rem)])
    plsc.subcore_barrier()

    per_tile = n_chunks * CH
    base_st = (c * NS + s) * (2 * per_tile)
    base_d = (c * NS + s) * per_tile

    def start_st(g, r):
        pltpu.async_copy(st_hbm.at[pl.ds(base_st + g * (2 * CH), 2 * CH)],
                         stbs[r], tsems[r])

    def wait_st(g, r):
        pltpu.make_async_copy(st_hbm.at[pl.ds(base_st + g * (2 * CH),
                                              2 * CH)],
                              stbs[r], tsems[r]).wait()

    def start_gather(g, r, b):
        pltpu.async_copy(
            sup_hbm.at[stbs[r].at[pl.ds(0, CH)]], rows.at[b], gsems[b])

    start_st(0, 0)
    start_st(1, 1)
    start_st(2, 2)
    wait_st(0, 0)
    start_gather(0, 0, 0)
    wait_st(1, 1)
    start_gather(1, 1, 1)

    def six_body(h):
        for k in range(6):
            g = h * 6 + k
            b = k % 2
            r = k % 3
            pltpu.make_async_copy(sup_hbm.at[stbs[r].at[pl.ds(0, CH)]],
                                  rows.at[b], gsems[b]).wait()
            pltpu.async_copy(dst_hbm.at[pl.ds(base_d + g * CH, CH)],
                             dstbs[b], dsems[b])
            @pl.when(g >= 1)
            def _drain():
                pltpu.make_async_copy(srows, accum.at[dstbs[1 - b]],
                                      ssem).wait()

            def grp(g2):
                tv = stbs[r][pl.ds(CH + g2 * 16, 16)]
                a16 = plsc.load_gather(alpha_v, [tv])
                for j in range(16):
                    asp = _splat(a16, j)
                    e = g2 * 16 + j
                    for v in range(HALF // 16):
                        srows[e, pl.ds(v * 16, 16)] = (
                            rows[b, e, pl.ds(v * 16, 16)] * asp)
                return None
            pl.loop(0, CH // 16)(grp)

            @pl.when(g + 3 < n_chunks)
            def _st_prefetch():
                start_st(g + 3, r)
            @pl.when(g + 2 < n_chunks)
            def _g_prefetch():
                wait_st(g + 2, (k + 2) % 3)
                start_gather(g + 2, (k + 2) % 3, b)
            pltpu.make_async_copy(dst_hbm.at[pl.ds(base_d + g * CH, CH)],
                                  dstbs[b], dsems[b]).wait()
            pltpu.async_copy(srows, accum.at[dstbs[b]], ssem,
                             add=True)
        return None
    pl.loop(0, n_chunks // 6)(six_body)

    lastb = (n_chunks - 1) % 2
    pltpu.make_async_copy(srows, accum.at[dstbs[lastb]], ssem).wait()

    plsc.subcore_barrier()
    pltpu.sync_copy(accum.at[pl.ds(s * STRIDE, STRIDE)],
                    out_hbm.at[c, pl.ds(s * STRIDE, STRIDE)])

    @pl.when(s == 0)
    def _copy_tail():
        pltpu.sync_copy(accum.at[pl.ds(NS * STRIDE, TAIL)],
                        out_hbm.at[c, pl.ds(NS * STRIDE, TAIL)])


def _make_edge_call(n_chunks):
    mesh = plsc.VectorSubcoreMesh(core_axis_name="c", subcore_axis_name="s",
                                  num_cores=NC, num_subcores=NS)
    return functools.partial(
        pl.kernel,
        out_type=jax.ShapeDtypeStruct((NC, N_ENT, HALF), jnp.float32),
        mesh=mesh,
        compiler_params=pltpu.CompilerParams(needs_layout_passes=False),
        scratch_types=[
            pltpu.VMEM_SHARED((ACC_ROWS, HALF), jnp.float32),
            pltpu.VMEM((2, CH, HALF), jnp.float32),
            pltpu.VMEM((CH, HALF), jnp.float32),
            pltpu.VMEM((2 * CH,), jnp.int32),
            pltpu.VMEM((2 * CH,), jnp.int32),
            pltpu.VMEM((2 * CH,), jnp.int32),
            pltpu.VMEM((CH,), jnp.int32),
            pltpu.VMEM((CH,), jnp.int32),
            pltpu.VMEM((208,), jnp.float32),
            pltpu.SemaphoreType.DMA,
            pltpu.SemaphoreType.DMA,
            pltpu.SemaphoreType.DMA,
            pltpu.SemaphoreType.DMA,
            pltpu.SemaphoreType.DMA,
            pltpu.SemaphoreType.DMA,
            pltpu.SemaphoreType.DMA,
            pltpu.SemaphoreType.DMA,
        ],
    )(functools.partial(_edge_body, n_chunks))


GB = 128


def _gather_body(x3_hbm, rel_tab_hbm, e1_hbm, e2_hbm, rel_hbm,
                 g1_hbm, g2_hbm, gr_hbm, idxb, rowsb, gsem):
    c = lax.axis_index("c")
    s = lax.axis_index("s")
    wid = s * NC + c
    base = wid * GB

    for idx_hbm, tab, out_hbm in ((e1_hbm, x3_hbm, g1_hbm),
                                  (e2_hbm, x3_hbm, g2_hbm),
                                  (rel_hbm, rel_tab_hbm, gr_hbm)):
        pltpu.sync_copy(idx_hbm.at[pl.ds(base, GB)], idxb)
        pltpu.async_copy(tab.at[idxb], rowsb, gsem).wait()
        pltpu.sync_copy(rowsb, out_hbm.at[pl.ds(base, GB)])


def _make_gather_call(batch):
    mesh = plsc.VectorSubcoreMesh(core_axis_name="c", subcore_axis_name="s",
                                  num_cores=NC, num_subcores=NS)
    osd = jax.ShapeDtypeStruct((batch, EMB), jnp.float32)
    return functools.partial(
        pl.kernel,
        out_type=(osd, osd, osd),
        mesh=mesh,
        scratch_types=[
            pltpu.VMEM((GB,), jnp.int32),
            pltpu.VMEM((GB, EMB), jnp.float32),
            pltpu.SemaphoreType.DMA,
        ],
    )(_gather_body)


def _mm_first_body(emb_ref, w_ref, out_ref):
    y = jnp.dot(emb_ref[...], w_ref[...], preferred_element_type=jnp.float32)
    out_ref[0] = y[:, :HALF]
    out_ref[1] = y[:, HALF:]


def _layer_body(agg_ref, b_ref, w_ref, out_ref):
    x = jnp.concatenate([agg_ref[0], agg_ref[1]], axis=1) + b_ref[...]
    m = jnp.mean(x, axis=0)
    v = jnp.mean(jnp.square(x), axis=0) - jnp.square(m)
    y = jnp.tanh((x - m) * lax.rsqrt(v + 1e-5))
    z = jnp.dot(y, w_ref[...], preferred_element_type=jnp.float32)
    out_ref[0] = z[:, :HALF]
    out_ref[1] = z[:, HALF:]


def _final_body(agg_ref, b_ref, out_ref):
    x = jnp.concatenate([agg_ref[0], agg_ref[1]], axis=1) + b_ref[...]
    m = jnp.mean(x, axis=0)
    v = jnp.mean(jnp.square(x), axis=0) - jnp.square(m)
    out_ref[...] = jnp.tanh((x - m) * lax.rsqrt(v + 1e-5))


def _score_body(g1_ref, g2_ref, gr_ref, out_ref):
    srow = jnp.sum(g1_ref[...] * g2_ref[...] * gr_ref[...], axis=1)
    out_ref[...] = 1.0 / (1.0 + jnp.exp(-srow))


def _tc(body, out_shape):
    return pl.pallas_call(body, out_shape=out_shape)


def kernel(e1, rel, e2, X, edge_index, edge_type, num_nodes,
           emb_e, W1, b1, alpha1, W2, b2, alpha2, W3, b3, alpha3, emb_rel):
    n = emb_e.shape[0]
    assert n == N_ENT
    e_dir = 2 * edge_index.shape[1]
    per_tile, e_pad = _edge_counts(e_dir)
    n_chunks = per_tile // CH

    row = edge_index[0].astype(jnp.int32)
    col = edge_index[1].astype(jnp.int32)
    et = edge_type.astype(jnp.int32)
    pad = e_pad - e_dir
    dst_dir = jnp.pad(jnp.concatenate([row, col]), (0, pad))
    src_dir = jnp.pad(jnp.concatenate([col, row]), (0, pad))
    typ_dir = jnp.pad(jnp.concatenate([et, et]), (0, pad))
    src2 = jnp.stack([src_dir, src_dir + N_ENT])
    src_blk = src2.reshape(NC, NS, n_chunks, CH)
    typ_blk = jnp.broadcast_to(typ_dir.reshape(1, NS, n_chunks, CH),
                               (NC, NS, n_chunks, CH))
    st_pack = jnp.concatenate([src_blk, typ_blk], axis=3).reshape(-1)
    dst2 = jnp.broadcast_to(dst_dir.reshape(1, NS * n_chunks * CH),
                            (NC, NS * n_chunks * CH)).reshape(-1)
    alpha_pad = [jnp.pad(a.astype(jnp.float32)[:, 0], (0, 208 - a.shape[0]))
                 for a in (alpha1, alpha2, alpha3)]

    edge_call = _make_edge_call(n_chunks)

    def sc_pass(sup, a16):
        return edge_call(sup.reshape(2 * N_ENT, HALF), st_pack, dst2, a16)

    sup = _tc(_mm_first_body,
              jax.ShapeDtypeStruct((2, N_ENT, HALF), jnp.float32))(emb_e, W1)
    agg = sc_pass(sup, alpha_pad[0])
    sup = _tc(_layer_body,
              jax.ShapeDtypeStruct((2, N_ENT, HALF), jnp.float32))(agg, b1, W2)
    agg = sc_pass(sup, alpha_pad[1])
    sup = _tc(_layer_body,
              jax.ShapeDtypeStruct((2, N_ENT, HALF), jnp.float32))(agg, b2, W3)
    agg = sc_pass(sup, alpha_pad[2])
    x3 = _tc(_final_body,
             jax.ShapeDtypeStruct((N_ENT, EMB), jnp.float32))(agg, b3)

    batch = e1.shape[0]
    g1, g2, gr = _make_gather_call(batch)(
        x3, emb_rel, e1.astype(jnp.int32), e2.astype(jnp.int32),
        rel.astype(jnp.int32))
    pred = _tc(_score_body,
               jax.ShapeDtypeStruct((batch,), jnp.float32))(g1, g2, gr)
    return pred


# speedup vs baseline: 12.9212x; 2.0542x over previous
import functools

import jax
import jax.numpy as jnp
from jax import lax
from jax.experimental import pallas as pl
from jax.experimental.pallas import tpu as pltpu
from jax.experimental.pallas import tpu_sc as plsc

N_ENT = 10000
N_REL = 200
INIT_EMB = 128
EMB = 256
HALF = 128
NC = 2
NS = 16
CH = 80
ACC_ROWS = N_ENT
STRIDE = 624
ZSPAN = ACC_ROWS - 15 * STRIDE
TAIL = N_ENT - NS * STRIDE


def _edge_counts(n_edges_dir):
    per_tile = -(-n_edges_dir // NS)
    per_tile = -(-per_tile // (4 * CH)) * (4 * CH)
    return per_tile, per_tile * NS


def _splat(vec, j):
    return lax.gather(
        vec, jnp.full((16, 1), j, jnp.int32),
        lax.GatherDimensionNumbers(offset_dims=(), collapsed_slice_dims=(0,),
                                   start_index_map=(0,)),
        (1,), mode=lax.GatherScatterMode.PROMISE_IN_BOUNDS)


def _edge_body(n_chunks, sup_hbm, st_hbm, dst_hbm, alpha_hbm,
               out_hbm, accum, rows, stb0, stb1, stb2, stb3,
               dstb0, dstb1, dstb2, dstb3, alpha_v,
               gsem0, gsem1, gsem2, gsem3, ssem0, ssem1, ssem2, ssem3,
               tsem0, tsem1, tsem2, tsem3, dsem0, dsem1, dsem2, dsem3):
    c = lax.axis_index("c")
    s = lax.axis_index("s")
    gsems = (gsem0, gsem1, gsem2, gsem3)
    ssems = (ssem0, ssem1, ssem2, ssem3)
    tsems = (tsem0, tsem1, tsem2, tsem3)
    dsems = (dsem0, dsem1, dsem2, dsem3)
    stbs = (stb0, stb1, stb2, stb3)
    dstbs = (dstb0, dstb1, dstb2, dstb3)
    pltpu.sync_copy(alpha_hbm, alpha_v)

    zb = jnp.zeros((16,), jnp.float32)
    for r in range(CH):
        for v in range(HALF // 16):
            rows[0, r, pl.ds(v * 16, 16)] = zb
    zbase = s * STRIDE
    nfull = ZSPAN // CH
    rem = ZSPAN - nfull * CH

    def zfill(g):
        pltpu.sync_copy(rows.at[0], accum.at[pl.ds(zbase + g * CH, CH)])
        return None
    pl.loop(0, nfull)(zfill)
    if rem:
        pltpu.sync_copy(rows.at[0, pl.ds(0, rem)],
                        accum.at[pl.ds(zbase + nfull * CH, rem)])
    plsc.subcore_barrier()

    per_tile = n_chunks * CH
    base_st = (c * NS + s) * (2 * per_tile)
    base_d = (c * NS + s) * per_tile

    def start_st(g, q):
        pltpu.async_copy(st_hbm.at[pl.ds(base_st + g * (2 * CH), 2 * CH)],
                         stbs[q], tsems[q])

    def wait_st(g, q):
        pltpu.make_async_copy(st_hbm.at[pl.ds(base_st + g * (2 * CH),
                                              2 * CH)],
                              stbs[q], tsems[q]).wait()

    def start_gather(g, q):
        pltpu.async_copy(
            sup_hbm.at[stbs[q].at[pl.ds(0, CH)]], rows.at[q], gsems[q])

    def wait_scatter(q):
        pltpu.make_async_copy(rows.at[q], accum.at[dstbs[q]],
                              ssems[q]).wait()

    start_st(0, 0)
    start_st(1, 1)
    start_st(2, 2)
    wait_st(0, 0)
    start_gather(0, 0)
    wait_st(1, 1)
    start_gather(1, 1)

    def quad_body(h):
        for k in range(4):
            g = h * 4 + k
            q = k
            pltpu.make_async_copy(sup_hbm.at[stbs[q].at[pl.ds(0, CH)]],
                                  rows.at[q], gsems[q]).wait()
            @pl.when(g + 3 < n_chunks)
            def _st_prefetch():
                start_st(g + 3, (k + 3) % 4)
            @pl.when(g >= 2)
            def _drain():
                wait_scatter((k + 2) % 4)
            @pl.when(g + 2 < n_chunks)
            def _g_prefetch():
                wait_st(g + 2, (k + 2) % 4)
                start_gather(g + 2, (k + 2) % 4)
            pltpu.async_copy(dst_hbm.at[pl.ds(base_d + g * CH, CH)],
                             dstbs[q], dsems[q])

            def grp(g2):
                tv = stbs[q][pl.ds(CH + g2 * 16, 16)]
                a16 = plsc.load_gather(alpha_v, [tv])
                for j in range(16):
                    asp = _splat(a16, j)
                    e = g2 * 16 + j
                    for v in range(HALF // 16):
                        rows[q, e, pl.ds(v * 16, 16)] = (
                            rows[q, e, pl.ds(v * 16, 16)] * asp)
                return None
            pl.loop(0, CH // 16)(grp)

            pltpu.make_async_copy(dst_hbm.at[pl.ds(base_d + g * CH, CH)],
                                  dstbs[q], dsems[q]).wait()
            pltpu.async_copy(rows.at[q], accum.at[dstbs[q]], ssems[q],
                             add=True)
        return None
    pl.loop(0, n_chunks // 4)(quad_body)

    wait_scatter((n_chunks - 2) % 4)
    wait_scatter((n_chunks - 1) % 4)

    plsc.subcore_barrier()
    pltpu.sync_copy(accum.at[pl.ds(s * STRIDE, STRIDE)],
                    out_hbm.at[c, pl.ds(s * STRIDE, STRIDE)])

    @pl.when(s == 0)
    def _copy_tail():
        pltpu.sync_copy(accum.at[pl.ds(NS * STRIDE, TAIL)],
                        out_hbm.at[c, pl.ds(NS * STRIDE, TAIL)])


def _make_edge_call(n_chunks):
    mesh = plsc.VectorSubcoreMesh(core_axis_name="c", subcore_axis_name="s",
                                  num_cores=NC, num_subcores=NS)
    return functools.partial(
        pl.kernel,
        out_type=jax.ShapeDtypeStruct((NC, N_ENT, HALF), jnp.float32),
        mesh=mesh,
        compiler_params=pltpu.CompilerParams(needs_layout_passes=False),
        scratch_types=[
            pltpu.VMEM_SHARED((ACC_ROWS, HALF), jnp.float32),
            pltpu.VMEM((4, CH, HALF), jnp.float32),
            pltpu.VMEM((2 * CH,), jnp.int32),
            pltpu.VMEM((2 * CH,), jnp.int32),
            pltpu.VMEM((2 * CH,), jnp.int32),
            pltpu.VMEM((2 * CH,), jnp.int32),
            pltpu.VMEM((CH,), jnp.int32),
            pltpu.VMEM((CH,), jnp.int32),
            pltpu.VMEM((CH,), jnp.int32),
            pltpu.VMEM((CH,), jnp.int32),
            pltpu.VMEM((208,), jnp.float32),
        ] + [pltpu.SemaphoreType.DMA] * 16,
    )(functools.partial(_edge_body, n_chunks))


GB = 128


def _gather_body(x3_hbm, rel_tab_hbm, e1_hbm, e2_hbm, rel_hbm,
                 g1_hbm, g2_hbm, gr_hbm, idxb, rowsb, gsem):
    c = lax.axis_index("c")
    s = lax.axis_index("s")
    wid = s * NC + c
    base = wid * GB

    for idx_hbm, tab, out_hbm in ((e1_hbm, x3_hbm, g1_hbm),
                                  (e2_hbm, x3_hbm, g2_hbm),
                                  (rel_hbm, rel_tab_hbm, gr_hbm)):
        pltpu.sync_copy(idx_hbm.at[pl.ds(base, GB)], idxb)
        pltpu.async_copy(tab.at[idxb], rowsb, gsem).wait()
        pltpu.sync_copy(rowsb, out_hbm.at[pl.ds(base, GB)])


def _make_gather_call(batch):
    mesh = plsc.VectorSubcoreMesh(core_axis_name="c", subcore_axis_name="s",
                                  num_cores=NC, num_subcores=NS)
    osd = jax.ShapeDtypeStruct((batch, EMB), jnp.float32)
    return functools.partial(
        pl.kernel,
        out_type=(osd, osd, osd),
        mesh=mesh,
        scratch_types=[
            pltpu.VMEM((GB,), jnp.int32),
            pltpu.VMEM((GB, EMB), jnp.float32),
            pltpu.SemaphoreType.DMA,
        ],
    )(_gather_body)


def _mm_first_body(emb_ref, w_ref, out_ref):
    y = jnp.dot(emb_ref[...], w_ref[...], preferred_element_type=jnp.float32)
    out_ref[0] = y[:, :HALF]
    out_ref[1] = y[:, HALF:]


def _layer_body(agg_ref, b_ref, w_ref, out_ref):
    x = jnp.concatenate([agg_ref[0], agg_ref[1]], axis=1) + b_ref[...]
    m = jnp.mean(x, axis=0)
    v = jnp.mean(jnp.square(x), axis=0) - jnp.square(m)
    y = jnp.tanh((x - m) * lax.rsqrt(v + 1e-5))
    z = jnp.dot(y, w_ref[...], preferred_element_type=jnp.float32)
    out_ref[0] = z[:, :HALF]
    out_ref[1] = z[:, HALF:]


def _final_body(agg_ref, b_ref, out_ref):
    x = jnp.concatenate([agg_ref[0], agg_ref[1]], axis=1) + b_ref[...]
    m = jnp.mean(x, axis=0)
    v = jnp.mean(jnp.square(x), axis=0) - jnp.square(m)
    out_ref[...] = jnp.tanh((x - m) * lax.rsqrt(v + 1e-5))


def _score_body(g1_ref, g2_ref, gr_ref, out_ref):
    srow = jnp.sum(g1_ref[...] * g2_ref[...] * gr_ref[...], axis=1)
    out_ref[...] = 1.0 / (1.0 + jnp.exp(-srow))


def _tc(body, out_shape):
    return pl.pallas_call(body, out_shape=out_shape)


def kernel(e1, rel, e2, X, edge_index, edge_type, num_nodes,
           emb_e, W1, b1, alpha1, W2, b2, alpha2, W3, b3, alpha3, emb_rel):
    n = emb_e.shape[0]
    assert n == N_ENT
    e_dir = 2 * edge_index.shape[1]
    per_tile, e_pad = _edge_counts(e_dir)
    n_chunks = per_tile // CH

    row = edge_index[0].astype(jnp.int32)
    col = edge_index[1].astype(jnp.int32)
    et = edge_type.astype(jnp.int32)
    pad = e_pad - e_dir
    dst_dir = jnp.pad(jnp.concatenate([row, col]), (0, pad))
    src_dir = jnp.pad(jnp.concatenate([col, row]), (0, pad))
    typ_dir = jnp.pad(jnp.concatenate([et, et]), (0, pad))
    src2 = jnp.stack([src_dir, src_dir + N_ENT])
    src_blk = src2.reshape(NC, NS, n_chunks, CH)
    typ_blk = jnp.broadcast_to(typ_dir.reshape(1, NS, n_chunks, CH),
                               (NC, NS, n_chunks, CH))
    st_pack = jnp.concatenate([src_blk, typ_blk], axis=3).reshape(-1)
    dst2 = jnp.broadcast_to(dst_dir.reshape(1, NS * n_chunks * CH),
                            (NC, NS * n_chunks * CH)).reshape(-1)
    alpha_pad = [jnp.pad(a.astype(jnp.float32)[:, 0], (0, 208 - a.shape[0]))
                 for a in (alpha1, alpha2, alpha3)]

    edge_call = _make_edge_call(n_chunks)

    def sc_pass(sup, a16):
        return edge_call(sup.reshape(2 * N_ENT, HALF), st_pack, dst2, a16)

    sup = _tc(_mm_first_body,
              jax.ShapeDtypeStruct((2, N_ENT, HALF), jnp.float32))(emb_e, W1)
    agg = sc_pass(sup, alpha_pad[0])
    sup = _tc(_layer_body,
              jax.ShapeDtypeStruct((2, N_ENT, HALF), jnp.float32))(agg, b1, W2)
    agg = sc_pass(sup, alpha_pad[1])
    sup = _tc(_layer_body,
              jax.ShapeDtypeStruct((2, N_ENT, HALF), jnp.float32))(agg, b2, W3)
    agg = sc_pass(sup, alpha_pad[2])
    x3 = _tc(_final_body,
             jax.ShapeDtypeStruct((N_ENT, EMB), jnp.float32))(agg, b3)

    batch = e1.shape[0]
    g1, g2, gr = _make_gather_call(batch)(
        x3, emb_rel, e1.astype(jnp.int32), e2.astype(jnp.int32),
        rel.astype(jnp.int32))
    pred = _tc(_score_body,
               jax.ShapeDtypeStruct((batch,), jnp.float32))(g1, g2, gr)
    return pred
